# baseline (speedup 1.0000x reference)
import functools

import jax
import jax.numpy as jnp
from jax.experimental import pallas as pl

_N_CHUNKS = 4


def _sage_kernel(x_ref, *refs, block_m, d_in):
    adj_refs = refs[:_N_CHUNKS]
    w_ref, b_ref, out_ref = refs[_N_CHUNKS:]
    i = pl.program_id(0)
    n = x_ref.shape[0]
    chunk = n // _N_CHUNKS
    s = None
    deg = None
    for c, adj_ref in enumerate(adj_refs):
        af = adj_ref[...].astype(jnp.float32)
        xc = x_ref[pl.ds(c * chunk, chunk), :]
        sc = jnp.dot(af, xc, preferred_element_type=jnp.float32)
        dc = jnp.sum(af, axis=1, keepdims=True)
        s = sc if s is None else s + sc
        deg = dc if deg is None else deg + dc
    x1 = s / deg
    xr = x_ref[pl.ds(i * block_m, block_m), :]
    b = b_ref[...]
    for k in range(out_ref.shape[0]):
        w1 = w_ref[k, :d_in, :]
        w2 = w_ref[k, d_in:, :]
        out_ref[k] = (
            jnp.dot(x1, w1, preferred_element_type=jnp.float32)
            + jnp.dot(xr, w2, preferred_element_type=jnp.float32)
            + b
        )


def kernel(g, x, adj, W, b):
    n, d_in = x.shape
    k3, two_d, d_out = W.shape
    block_m = 512
    chunk = n // _N_CHUNKS
    grid = (n // block_m,)
    body = functools.partial(_sage_kernel, block_m=block_m, d_in=d_in)

    def adj_spec(c):
        return pl.BlockSpec((block_m, chunk), lambda i, c=c: (i, c))

    out = pl.pallas_call(
        body,
        grid=grid,
        in_specs=[
            pl.BlockSpec((n, d_in), lambda i: (0, 0)),
            *[adj_spec(c) for c in range(_N_CHUNKS)],
            pl.BlockSpec((k3, two_d, d_out), lambda i: (0, 0, 0)),
            pl.BlockSpec((d_out,), lambda i: (0,)),
        ],
        out_specs=pl.BlockSpec((k3, block_m, d_out), lambda i: (0, i, 0)),
        out_shape=jax.ShapeDtypeStruct((k3, n, d_out), jnp.float32),
    )(x, *([adj] * _N_CHUNKS), W, b)
    return out
